# baseline (speedup 1.0000x reference)
import functools

import jax
import jax.numpy as jnp
from jax import lax
from jax.experimental import pallas as pl
from jax.experimental.pallas import tpu as pltpu
from jax.experimental.pallas import tpu_sc as plsc

_INF = 1.0e7
_EOS_ID = 2
_LANES = 16
_K = 16
_CHUNK = 50000
_GROUP = 25
_CROWS = 64


def _make_sc_topk(n_rows, vocab, n_drafts):
    n_prompts = n_rows // n_drafts
    n_chunks = vocab // _CHUNK
    assert n_chunks * _CHUNK == vocab and n_chunks % 2 == 0
    groups_per_chunk = _CHUNK // (_GROUP * _LANES)
    assert groups_per_chunk * _GROUP * _LANES == _CHUNK

    mesh = plsc.VectorSubcoreMesh(core_axis_name="c", subcore_axis_name="s")

    @functools.partial(
        pl.kernel,
        out_type=(
            jax.ShapeDtypeStruct((n_prompts, n_drafts, _K), jnp.float32),
            jax.ShapeDtypeStruct((n_prompts, n_drafts, _K), jnp.int32),
        ),
        mesh=mesh,
        compiler_params=pltpu.CompilerParams(needs_layout_passes=False,
                                             use_tc_tiling_on_sc=False),
        scratch_types=[
            pltpu.VMEM((_CHUNK,), jnp.float32),
            pltpu.VMEM((_CHUNK,), jnp.float32),
            pltpu.VMEM((_CROWS * _LANES,), jnp.float32),
            pltpu.VMEM((_CROWS * _LANES,), jnp.int32),
            pltpu.VMEM((_K,), jnp.float32),
            pltpu.VMEM((_K,), jnp.int32),
            pltpu.SemaphoreType.DMA,
            pltpu.SemaphoreType.DMA,
        ],
    )
    def sc_topk(probs_hbm, out_val, out_idx, buf0, buf1, cb_val, cb_idx,
                ov, oi, sem0, sem1):
        wid = lax.axis_index("s") * 2 + lax.axis_index("c")
        iota = lax.iota(jnp.int32, _LANES)
        bufs = (buf0, buf1)
        sems = (sem0, sem1)

        def src(d, c):
            return probs_hbm.at[wid * n_drafts + d, pl.ds(c * _CHUNK, _CHUNK)]

        def start(d, c, par):
            pltpu.async_copy(src(d, c), bufs[par], sems[par])

        def wait(d, c, par):
            pltpu.make_async_copy(src(d, c), bufs[par], sems[par]).wait()

        def any_lane(mask):
            cnt = plsc.all_reduce_population_count(mask)
            return lax.squeeze(lax.slice(cnt, (0,), (1,)), (0,)) > 0

        def elem(vec, i):
            return lax.squeeze(lax.slice(vec, (i,), (i + 1,)), (0,))

        def merge_vec(rv, ri, v, vi):
            sv, si = plsc.sort_key_val(v, vi, descending=True)
            take = sv > rv
            hi_v = jnp.where(take, sv, rv)
            hi_i = jnp.where(take, si, ri)
            nv, ni = plsc.sort_key_val(hi_v, hi_i, descending=False)
            return nv, ni

        def consolidate(cc):
            r_val, r_idx, thresh, p = cc
            ps, _ = plsc.sort_key_val(p, p, descending=False)
            maxp = elem(ps, _LANES - 1)

            def body(e, rc):
                rv, ri = rc
                v = cb_val[pl.ds(e * _LANES, _LANES)]
                vi = cb_idx[pl.ds(e * _LANES, _LANES)]
                valid = e < p
                v = jnp.where(valid, v, 0.0)
                vi = jnp.where(valid, vi, 0)
                return merge_vec(rv, ri, v, vi)

            r_val, r_idx = lax.fori_loop(0, maxp, body, (r_val, r_idx))
            thresh = jnp.maximum(thresh, elem(r_val, 0))
            return (r_val, r_idx, thresh, jnp.zeros((_LANES,), jnp.int32))

        start(0, 0, 0)

        def row_body(d, _):
            carry = (jnp.zeros((_K,), jnp.float32),
                     jnp.zeros((_K,), jnp.int32),
                     jnp.float32(0.0),
                     jnp.zeros((_LANES,), jnp.int32))
            for c in range(n_chunks):
                par = c % 2
                wait(d, c, par)
                if c + 1 < n_chunks:
                    start(d, c + 1, (c + 1) % 2)
                else:
                    start(jnp.minimum(d + 1, n_drafts - 1), 0, 0)
                buf = bufs[par]

                if c == 0:
                    r_val, r_idx, thresh, p = carry
                    for k in range(_GROUP):
                        v = buf[pl.ds(k * _LANES, _LANES)]
                        vi = (k * _LANES) + iota
                        r_val, r_idx = merge_vec(r_val, r_idx, v, vi)
                    carry = (r_val, r_idx, elem(r_val, 0), p)

                def group_body(g, carry, c=c, buf=buf):
                    r_val, r_idx, thresh, p = carry
                    gb = g * (_GROUP * _LANES)
                    vecs = [buf[pl.ds(gb + k * _LANES, _LANES)]
                            for k in range(_GROUP)]
                    while len(vecs) > 1:
                        nxt = [jnp.maximum(vecs[i], vecs[i + 1])
                               for i in range(0, len(vecs) - 1, 2)]
                        if len(vecs) % 2:
                            nxt.append(vecs[-1])
                        vecs = nxt
                    m = vecs[0]

                    def on_hit(cc):
                        r_val, r_idx, thresh, p = cc
                        for k in range(_GROUP):
                            v = buf[pl.ds(gb + k * _LANES, _LANES)]
                            mask = v > thresh
                            addr = p * _LANES + iota
                            vi = (c * _CHUNK + gb + k * _LANES) + iota
                            plsc.store_scatter(cb_val, [addr], v, mask=mask)
                            plsc.store_scatter(cb_idx, [addr], vi, mask=mask)
                            p = p + jnp.where(mask, 1, 0)
                        cc = (r_val, r_idx, thresh, p)
                        return lax.cond(any_lane(p > _CROWS - _GROUP - 1),
                                        consolidate, lambda x: x, cc)

                    return lax.cond(any_lane(m > thresh), on_hit,
                                    lambda x: x, carry)

                g_lo = 1 if c == 0 else 0
                carry = lax.fori_loop(g_lo, groups_per_chunk, group_body, carry)

            r_val, r_idx, _, _ = consolidate(carry)
            ov[...] = r_val
            oi[...] = r_idx
            pltpu.sync_copy(ov, out_val.at[wid, d])
            pltpu.sync_copy(oi, out_idx.at[wid, d])
            return 0

        lax.fori_loop(0, n_drafts, row_body, 0)
        wait(n_drafts - 1, 0, 0)

    return sc_topk


def kernel(probs, still_prompt, is_first, cur_pos, n_token_consider,
           n_token_sample, alive_seq, alive_log_probs, fin_seq, fin_log_probs):
    n_prompts, n_drafts = alive_log_probs.shape
    vocab = probs.shape[-1]

    sc_topk = _make_sc_topk(probs.shape[0], vocab, n_drafts)
    cand_val, cand_idx = sc_topk(probs)

    order = jnp.argsort(cand_idx, axis=-1)
    cand_val = jnp.take_along_axis(cand_val, order, axis=-1)
    cand_idx = jnp.take_along_axis(cand_idx, order, axis=-1)

    scores = alive_log_probs[:, :, None] + jnp.log(cand_val)
    scores_flat = scores.reshape(n_prompts, n_drafts * _K)
    idx_flat = cand_idx.reshape(n_prompts, n_drafts * _K)

    topk_log_probs, pos = jax.lax.top_k(scores_flat, 2 * n_drafts)
    topk_beam_id = pos // _K
    topk_idx = jnp.take_along_axis(idx_flat, pos, axis=1)

    topk_finished = topk_idx == _EOS_ID
    alive_scores = topk_log_probs + jnp.where(topk_finished, -_INF, 0.0)
    _, alive_sel = jax.lax.top_k(alive_scores, n_drafts)
    ids = jnp.take_along_axis(topk_beam_id, alive_sel, axis=1)

    ids = jnp.where(is_first[:, None], jnp.zeros_like(ids), ids)
    ids = jnp.where(still_prompt[:, None],
                    jnp.broadcast_to(jnp.arange(n_drafts, dtype=ids.dtype),
                                     (n_prompts, n_drafts)),
                    ids)
    return ids


# speedup vs baseline: 1.3914x; 1.3914x over previous
import functools

import jax
import jax.numpy as jnp
from jax import lax
from jax.experimental import pallas as pl
from jax.experimental.pallas import tpu as pltpu
from jax.experimental.pallas import tpu_sc as plsc

_INF = 1.0e7
_EOS_ID = 2
_LANES = 16
_K = 16
_CHUNK = 50000
_GROUP = 25
_CROWS = 64
_ABLK = 5


def _make_sc_topk(n_rows, vocab, n_drafts):
    n_prompts = n_rows // n_drafts
    n_chunks = vocab // _CHUNK
    assert n_chunks * _CHUNK == vocab and n_chunks % 2 == 0
    groups_per_chunk = _CHUNK // (_GROUP * _LANES)
    assert groups_per_chunk * _GROUP * _LANES == _CHUNK

    mesh = plsc.VectorSubcoreMesh(core_axis_name="c", subcore_axis_name="s")

    @functools.partial(
        pl.kernel,
        out_type=(
            jax.ShapeDtypeStruct((n_prompts, n_drafts, _K), jnp.float32),
            jax.ShapeDtypeStruct((n_prompts, n_drafts, _K), jnp.int32),
        ),
        mesh=mesh,
        compiler_params=pltpu.CompilerParams(needs_layout_passes=False,
                                             use_tc_tiling_on_sc=False),
        scratch_types=[
            pltpu.VMEM((_CHUNK,), jnp.float32),
            pltpu.VMEM((_CHUNK,), jnp.float32),
            pltpu.VMEM((_CROWS * _LANES,), jnp.float32),
            pltpu.VMEM((_CROWS * _LANES,), jnp.int32),
            pltpu.VMEM((_K,), jnp.float32),
            pltpu.VMEM((_K,), jnp.int32),
            pltpu.SemaphoreType.DMA,
            pltpu.SemaphoreType.DMA,
        ],
    )
    def sc_topk(probs_hbm, out_val, out_idx, buf0, buf1, cb_val, cb_idx,
                ov, oi, sem0, sem1):
        wid = lax.axis_index("s") * 2 + lax.axis_index("c")
        iota = lax.iota(jnp.int32, _LANES)
        bufs = (buf0, buf1)
        sems = (sem0, sem1)

        def src(d, c):
            return probs_hbm.at[wid * n_drafts + d, pl.ds(c * _CHUNK, _CHUNK)]

        def start(d, c, par):
            pltpu.async_copy(src(d, c), bufs[par], sems[par])

        def wait(d, c, par):
            pltpu.make_async_copy(src(d, c), bufs[par], sems[par]).wait()

        def any_lane(mask):
            cnt = plsc.all_reduce_population_count(mask)
            return lax.squeeze(lax.slice(cnt, (0,), (1,)), (0,)) > 0

        def elem(vec, i):
            return lax.squeeze(lax.slice(vec, (i,), (i + 1,)), (0,))

        def merge_vec(rv, ri, v, vi):
            sv, si = plsc.sort_key_val(v, vi, descending=True)
            take = sv > rv
            hi_v = jnp.where(take, sv, rv)
            hi_i = jnp.where(take, si, ri)
            nv, ni = plsc.sort_key_val(hi_v, hi_i, descending=False)
            return nv, ni

        def consolidate(cc, trip):
            r_val, r_idx, thresh, p = cc

            def body(e, rc):
                rv, ri, th = rc
                v = cb_val[pl.ds(e * _LANES, _LANES)]
                vi = cb_idx[pl.ds(e * _LANES, _LANES)]
                valid = e < p
                v = jnp.where(valid, v, 0.0)
                vi = jnp.where(valid, vi, 0)
                rv, ri = merge_vec(rv, ri, v, vi)
                return (rv, ri, jnp.maximum(th, elem(rv, 0)))

            r_val, r_idx, thresh = lax.fori_loop(
                0, trip, body, (r_val, r_idx, thresh))
            p = jnp.where(trip > 0, jnp.zeros((_LANES,), jnp.int32), p)
            return (r_val, r_idx, thresh, p)

        start(0, 0, 0)

        def row_body(d, _):
            carry = (jnp.zeros((_K,), jnp.float32),
                     jnp.zeros((_K,), jnp.int32),
                     jnp.float32(0.0),
                     jnp.zeros((_LANES,), jnp.int32))
            for c in range(n_chunks):
                par = c % 2
                wait(d, c, par)
                if c + 1 < n_chunks:
                    start(d, c + 1, (c + 1) % 2)
                else:
                    start(jnp.minimum(d + 1, n_drafts - 1), 0, 0)
                buf = bufs[par]

                if c == 0:
                    r_val, r_idx, thresh, p = carry
                    for k in range(2 * _GROUP):
                        v = buf[pl.ds(k * _LANES, _LANES)]
                        vi = (k * _LANES) + iota
                        r_val, r_idx = merge_vec(r_val, r_idx, v, vi)
                    carry = (r_val, r_idx, elem(r_val, 0), p)

                def group_body(g, carry, c=c, buf=buf):
                    r_val, r_idx, thresh, p = carry
                    gb = g * (_GROUP * _LANES)
                    vecs = [buf[pl.ds(gb + k * _LANES, _LANES)]
                            for k in range(_GROUP)]
                    while len(vecs) > 1:
                        nxt = [jnp.maximum(vecs[i], vecs[i + 1])
                               for i in range(0, len(vecs) - 1, 2)]
                        if len(vecs) % 2:
                            nxt.append(vecs[-1])
                        vecs = nxt
                    m = vecs[0]

                    hit = any_lane(m > thresh)

                    def append_block(b, cc):
                        rv, ri, th, pp = cc
                        for k in range(_GROUP // _ABLK):
                            off = gb + (b * (_GROUP // _ABLK) + k) * _LANES
                            v = buf[pl.ds(off, _LANES)]
                            mask = v > th
                            addr = pp * _LANES + iota
                            vi = (c * _CHUNK) + off + iota
                            plsc.store_scatter(cb_val, [addr], v, mask=mask)
                            plsc.store_scatter(cb_idx, [addr], vi, mask=mask)
                            pp = pp + jnp.where(mask, 1, 0)
                        return (rv, ri, th, pp)

                    carry = lax.fori_loop(0, jnp.where(hit, _ABLK, 0),
                                          append_block, carry)
                    p_now = carry[3]
                    need = any_lane(p_now > _CROWS - _GROUP - 1)
                    return consolidate(carry, jnp.where(need, _CROWS, 0))

                g_lo = 2 if c == 0 else 0
                carry = lax.fori_loop(g_lo, groups_per_chunk, group_body, carry)

            ps, _ = plsc.sort_key_val(carry[3], carry[3], descending=False)
            r_val, r_idx, _, _ = consolidate(carry, elem(ps, _LANES - 1))
            si, sv = plsc.sort_key_val(r_idx, r_val, descending=False)
            ov[...] = sv
            oi[...] = si
            pltpu.sync_copy(ov, out_val.at[wid, d])
            pltpu.sync_copy(oi, out_idx.at[wid, d])
            return 0

        lax.fori_loop(0, n_drafts, row_body, 0)
        wait(n_drafts - 1, 0, 0)

    return sc_topk


def kernel(probs, still_prompt, is_first, cur_pos, n_token_consider,
           n_token_sample, alive_seq, alive_log_probs, fin_seq, fin_log_probs):
    n_prompts, n_drafts = alive_log_probs.shape
    vocab = probs.shape[-1]

    sc_topk = _make_sc_topk(probs.shape[0], vocab, n_drafts)
    cand_val, cand_idx = sc_topk(probs)

    scores = alive_log_probs[:, :, None] + jnp.log(cand_val)
    scores_flat = scores.reshape(n_prompts, n_drafts * _K)
    idx_flat = cand_idx.reshape(n_prompts, n_drafts * _K)

    topk_log_probs, pos = jax.lax.top_k(scores_flat, 2 * n_drafts)
    topk_beam_id = pos // _K

    oh_pos = (pos[:, :, None] == jnp.arange(n_drafts * _K)[None, None, :])
    fin_flat = (idx_flat == _EOS_ID).astype(jnp.float32)
    topk_finished = jnp.einsum("bkd,bd->bk", oh_pos.astype(jnp.float32),
                               fin_flat) > 0.5

    alive_scores = topk_log_probs + jnp.where(topk_finished, -_INF, 0.0)
    _, alive_sel = jax.lax.top_k(alive_scores, n_drafts)
    oh_sel = (alive_sel[:, :, None] == jnp.arange(2 * n_drafts)[None, None, :])
    ids = jnp.einsum("bkd,bd->bk", oh_sel.astype(jnp.float32),
                     topk_beam_id.astype(jnp.float32)).astype(jnp.int32)

    ids = jnp.where(is_first[:, None], jnp.zeros_like(ids), ids)
    ids = jnp.where(still_prompt[:, None],
                    jnp.broadcast_to(jnp.arange(n_drafts, dtype=ids.dtype),
                                     (n_prompts, n_drafts)),
                    ids)
    return ids
